# baseline (speedup 1.0000x reference)
import functools

import jax
import jax.numpy as jnp
from jax import lax
from jax.experimental import pallas as pl
from jax.experimental.pallas import tpu as pltpu
from jax.experimental.pallas import tpu_sc as plsc


def _make_gather(num_rows, F, D, NC, NS):
    NW = NC * NS
    N = num_rows // NW
    G = 16
    NG = N // G
    NSLOT = 3
    mesh = plsc.VectorSubcoreMesh(core_axis_name="c", subcore_axis_name="s")

    @functools.partial(
        pl.kernel,
        mesh=mesh,
        compiler_params=pltpu.CompilerParams(needs_layout_passes=False),
        out_type=jax.ShapeDtypeStruct((NW, 1, N * D), jnp.float32),
        scratch_types=[
            pltpu.VMEM((1, N), jnp.int32),
            pltpu.VMEM((NSLOT, 1, G * D), jnp.float32),
            pltpu.VMEM((NSLOT, G, D, 128), jnp.float32),
            [pltpu.SemaphoreType.DMA] * NSLOT,
            [pltpu.SemaphoreType.DMA] * NSLOT,
        ],
    )
    def gather_k(tbl, vh, out, vv, rows, blk, sems, osems):
        wid = lax.axis_index("s") * NC + lax.axis_index("c")
        pltpu.sync_copy(vh.at[wid], vv)
        iota16 = lax.iota(jnp.int32, 16)

        def fire(g, slot):
            @pl.when(g < NG)
            def _():
                vvec = vv[0, pl.ds(pl.multiple_of(g * G, 16), G)]
                for j in range(G):
                    w = vvec[j]
                    rb = pl.multiple_of(w >> 20, 8)
                    vb = pl.multiple_of(((w >> 7) & 0x1FFF) * 128, 128)
                    pltpu.async_copy(
                        tbl.at[pl.ds(rb, D), pl.ds(vb, 128)],
                        blk.at[slot, j],
                        sems[slot],
                    )

        def extract(g, slot):
            @pl.when(g >= NSLOT)
            def _():
                pltpu.make_async_copy(
                    rows.at[slot], out.at[wid, :, pl.ds(0, G * D)], osems[slot]
                ).wait()
            pltpu.make_async_copy(
                tbl.at[pl.ds(0, G * D), pl.ds(0, 128)], blk.at[slot], sems[slot]
            ).wait()
            vvec = vv[0, pl.ds(pl.multiple_of(g * G, 16), G)]
            for j in range(G):
                col = jnp.full((16,), vvec[j] & 0x7F, jnp.int32)
                vec = plsc.load_gather(blk.at[slot, j], [iota16, col])
                rows[slot, 0, pl.ds(j * D, D)] = vec
            pltpu.async_copy(
                rows.at[slot],
                out.at[wid, :, pl.ds(pl.multiple_of(g * G * D, 128), G * D)],
                osems[slot],
            )

        fire(0, 0)
        fire(1, 1)

        def body(i, carry):
            g = NSLOT * i
            fire(g + 2, 2)
            extract(g, 0)
            fire(g + 3, 0)
            extract(g + 1, 1)
            fire(g + 4, 1)
            extract(g + 2, 2)
            return carry

        lax.fori_loop(0, NG // NSLOT, body, 0)
        for t in range(NG % NSLOT):
            extract(NG - (NG % NSLOT) + t, t)
        for slot in range(NSLOT):
            pltpu.make_async_copy(
                rows.at[slot], out.at[wid, :, pl.ds(0, G * D)], osems[slot]
            ).wait()

    return gather_k, NW, N


def _mlp_body(emb_ref, W1_ref, b1_ref, W2_ref, b2_ref, W3_ref, b3_ref, out_ref):
    h = jnp.dot(emb_ref[...], W1_ref[...], preferred_element_type=jnp.float32)
    h = jnp.maximum(h + b1_ref[...][None, :], 0.0)
    h = jnp.dot(h, W2_ref[...], preferred_element_type=jnp.float32)
    h = jnp.maximum(h + b2_ref[...][None, :], 0.0)
    out = jnp.dot(h, W3_ref[...], preferred_element_type=jnp.float32)
    out_ref[...] = out + b3_ref[...][None, :]


def kernel(x, tables, W1, b1, W2, b2, W3, b3):
    B, F = x.shape
    _, V, D = tables.shape
    FD = F * D
    H = W1.shape[1]
    OUT = W3.shape[1]

    info = plsc.get_sparse_core_info()
    NC, NS = info.num_cores, info.num_subcores

    SPLITS = (3 * B // 4, B // 4)

    table_t = tables.transpose(0, 2, 1).reshape(FD, V)

    rb_hi = (jnp.arange(F, dtype=jnp.int32) * D) << 20
    packed = x | rb_hi[None, :]

    BLK = 512

    def mlp(emb, rows):
        return pl.pallas_call(
            _mlp_body,
            grid=(rows // BLK,),
            in_specs=[
                pl.BlockSpec((BLK, FD), lambda i: (i, 0)),
                pl.BlockSpec((FD, H), lambda i: (0, 0)),
                pl.BlockSpec((H,), lambda i: (0,)),
                pl.BlockSpec((H, H), lambda i: (0, 0)),
                pl.BlockSpec((H,), lambda i: (0,)),
                pl.BlockSpec((H, OUT), lambda i: (0, 0)),
                pl.BlockSpec((OUT,), lambda i: (0,)),
            ],
            out_specs=pl.BlockSpec((BLK, OUT), lambda i: (i, 0)),
            out_shape=jax.ShapeDtypeStruct((rows, OUT), jnp.float32),
        )(emb, W1, b1, W2, b2, W3, b3)

    outs = []
    off = 0
    for rows in SPLITS:
        gather_k, NW, N = _make_gather(rows * F, F, D, NC, NS)
        vh = packed[off:off + rows].reshape(NW, 1, N)
        emb = gather_k(table_t, vh).reshape(rows, FD)
        outs.append(mlp(emb, rows))
        off += rows
    return jnp.concatenate(outs, axis=0)


# speedup vs baseline: 1.0874x; 1.0874x over previous
import functools

import jax
import jax.numpy as jnp
from jax import lax
from jax.experimental import pallas as pl
from jax.experimental.pallas import tpu as pltpu
from jax.experimental.pallas import tpu_sc as plsc


def _make_gather(num_rows, F, D, NC, NS):
    NW = NC * NS
    N = num_rows // NW
    G = 16
    NG = N // G
    NSLOT = 3
    mesh = plsc.VectorSubcoreMesh(core_axis_name="c", subcore_axis_name="s")

    @functools.partial(
        pl.kernel,
        mesh=mesh,
        compiler_params=pltpu.CompilerParams(needs_layout_passes=False),
        out_type=jax.ShapeDtypeStruct((NW, 1, N * D), jnp.float32),
        scratch_types=[
            pltpu.VMEM((1, N), jnp.int32),
            pltpu.VMEM((NSLOT, 1, G * D), jnp.float32),
            pltpu.VMEM((NSLOT, G, D, 128), jnp.float32),
            [pltpu.SemaphoreType.DMA] * NSLOT,
            [pltpu.SemaphoreType.DMA] * NSLOT,
        ],
    )
    def gather_k(tbl, vh, out, vv, rows, blk, sems, osems):
        wid = lax.axis_index("s") * NC + lax.axis_index("c")
        pltpu.sync_copy(vh.at[wid], vv)
        iota16 = lax.iota(jnp.int32, 16)

        def fire(g, slot):
            @pl.when(g < NG)
            def _():
                vvec = vv[0, pl.ds(pl.multiple_of(g * G, 16), G)]
                for j in range(G):
                    w = vvec[j]
                    rb = pl.multiple_of(w >> 20, 8)
                    vb = pl.multiple_of(((w >> 7) & 0x1FFF) * 128, 128)
                    pltpu.async_copy(
                        tbl.at[pl.ds(rb, D), pl.ds(vb, 128)],
                        blk.at[slot, j],
                        sems[slot],
                    )

        def extract(g, slot):
            @pl.when(g >= NSLOT)
            def _():
                pltpu.make_async_copy(
                    rows.at[slot], out.at[wid, :, pl.ds(0, G * D)], osems[slot]
                ).wait()
            pltpu.make_async_copy(
                tbl.at[pl.ds(0, G * D), pl.ds(0, 128)], blk.at[slot], sems[slot]
            ).wait()
            vvec = vv[0, pl.ds(pl.multiple_of(g * G, 16), G)]
            for j in range(G):
                col = jnp.full((16,), vvec[j] & 0x7F, jnp.int32)
                vec = plsc.load_gather(blk.at[slot, j], [iota16, col])
                rows[slot, 0, pl.ds(j * D, D)] = vec
            pltpu.async_copy(
                rows.at[slot],
                out.at[wid, :, pl.ds(pl.multiple_of(g * G * D, 128), G * D)],
                osems[slot],
            )

        fire(0, 0)
        fire(1, 1)

        def body(i, carry):
            g = NSLOT * i
            fire(g + 2, 2)
            extract(g, 0)
            fire(g + 3, 0)
            extract(g + 1, 1)
            fire(g + 4, 1)
            extract(g + 2, 2)
            return carry

        lax.fori_loop(0, NG // NSLOT, body, 0)
        for t in range(NG % NSLOT):
            extract(NG - (NG % NSLOT) + t, t)
        for slot in range(NSLOT):
            pltpu.make_async_copy(
                rows.at[slot], out.at[wid, :, pl.ds(0, G * D)], osems[slot]
            ).wait()

    return gather_k, NW, N


def _mlp_body(emb_ref, W1_ref, b1_ref, W2_ref, b2_ref, W3_ref, b3_ref, out_ref):
    h = jnp.dot(emb_ref[...], W1_ref[...], preferred_element_type=jnp.float32)
    h = jnp.maximum(h + b1_ref[...][None, :], 0.0)
    h = jnp.dot(h, W2_ref[...], preferred_element_type=jnp.float32)
    h = jnp.maximum(h + b2_ref[...][None, :], 0.0)
    out = jnp.dot(h, W3_ref[...], preferred_element_type=jnp.float32)
    out_ref[...] = out + b3_ref[...][None, :]


def kernel(x, tables, W1, b1, W2, b2, W3, b3):
    B, F = x.shape
    _, V, D = tables.shape
    FD = F * D
    H = W1.shape[1]
    OUT = W3.shape[1]

    info = plsc.get_sparse_core_info()
    NC, NS = info.num_cores, info.num_subcores

    NSPLIT = 4
    HALF = B // NSPLIT
    gather_k, NW, N = _make_gather(HALF * F, F, D, NC, NS)

    table_t = tables.transpose(0, 2, 1).reshape(FD, V)

    rb_hi = (jnp.arange(F, dtype=jnp.int32) * D) << 20
    packed = x | rb_hi[None, :]

    BLK = 512
    grid = (HALF // BLK,)

    def mlp(emb):
        return pl.pallas_call(
            _mlp_body,
            grid=grid,
            in_specs=[
                pl.BlockSpec((BLK, FD), lambda i: (i, 0)),
                pl.BlockSpec((FD, H), lambda i: (0, 0)),
                pl.BlockSpec((H,), lambda i: (0,)),
                pl.BlockSpec((H, H), lambda i: (0, 0)),
                pl.BlockSpec((H,), lambda i: (0,)),
                pl.BlockSpec((H, OUT), lambda i: (0, 0)),
                pl.BlockSpec((OUT,), lambda i: (0,)),
            ],
            out_specs=pl.BlockSpec((BLK, OUT), lambda i: (i, 0)),
            out_shape=jax.ShapeDtypeStruct((HALF, OUT), jnp.float32),
        )(emb, W1, b1, W2, b2, W3, b3)

    outs = []
    for h in range(NSPLIT):
        vh = packed[h * HALF:(h + 1) * HALF].reshape(NW, 1, N)
        emb = gather_k(table_t, vh).reshape(HALF, FD)
        outs.append(mlp(emb))
    return jnp.concatenate(outs, axis=0)
